# baseline (speedup 1.0000x reference)
import jax
import jax.numpy as jnp
from jax.experimental import pallas as pl
from jax.experimental.pallas import tpu as pltpu

SIZE = 262144
EMBED_DIM = 128
BATCH = 4096
CHUNK = 8192
NCHUNK = SIZE // CHUNK
NBUF = 12


def _body(ptr_ref, q_hbm, emb_hbm, out_hbm, optr_ref,
          bufs, emb_buf, sem_in, sem_out, sem_emb):
    raw_ptr = ptr_ref[0]
    ptr = jnp.clip(raw_ptr, 0, SIZE - BATCH)
    optr_ref[0] = (raw_ptr + BATCH) % SIZE

    k0 = ptr // CHUNK

    def in_copy(i):
        c = ((k0 + i) % NCHUNK) * CHUNK
        s = i % NBUF
        return pltpu.make_async_copy(
            q_hbm.at[pl.ds(c, CHUNK)], bufs.at[s], sem_in.at[s])

    def out_copy(i):
        c = ((k0 + i) % NCHUNK) * CHUNK
        s = i % NBUF
        return pltpu.make_async_copy(
            bufs.at[s], out_hbm.at[pl.ds(c, CHUNK)], sem_out.at[s])

    emb_in = pltpu.make_async_copy(emb_hbm, emb_buf, sem_emb)
    emb_out = pltpu.make_async_copy(
        emb_buf, out_hbm.at[pl.ds(ptr, BATCH)], sem_emb)

    emb_in.start()
    for s in range(NBUF):
        in_copy(s).start()

    out_waited = set()

    def ensure_out(j):
        if j not in out_waited:
            out_copy(j).wait()
            out_waited.add(j)

    for i in range(NCHUNK):
        in_copy(i).wait()
        out_copy(i).start()
        if i == 1:
            ensure_out(0)
            ensure_out(1)
            emb_in.wait()
            emb_out.start()
        nxt = i + NBUF
        if nxt < NCHUNK:
            ensure_out(nxt - NBUF)
            in_copy(nxt).start()
    for j in range(NCHUNK):
        ensure_out(j)
    emb_out.wait()


def kernel(embed_queue, queue_ptr, embeddings):
    new_queue, new_ptr = pl.pallas_call(
        _body,
        in_specs=[
            pl.BlockSpec(memory_space=pltpu.SMEM),
            pl.BlockSpec(memory_space=pl.ANY),
            pl.BlockSpec(memory_space=pl.ANY),
        ],
        out_specs=[
            pl.BlockSpec(memory_space=pl.ANY),
            pl.BlockSpec(memory_space=pltpu.SMEM),
        ],
        out_shape=[
            jax.ShapeDtypeStruct((SIZE, EMBED_DIM), jnp.float32),
            jax.ShapeDtypeStruct((1,), jnp.int32),
        ],
        scratch_shapes=[
            pltpu.VMEM((NBUF, CHUNK, EMBED_DIM), jnp.float32),
            pltpu.VMEM((BATCH, EMBED_DIM), jnp.float32),
            pltpu.SemaphoreType.DMA((NBUF,)),
            pltpu.SemaphoreType.DMA((NBUF,)),
            pltpu.SemaphoreType.DMA,
        ],
    )(queue_ptr, embed_queue, embeddings)
    return new_queue, new_ptr


# speedup vs baseline: 1.0166x; 1.0166x over previous
import jax
import jax.numpy as jnp
from jax.experimental import pallas as pl
from jax.experimental.pallas import tpu as pltpu

SIZE = 262144
EMBED_DIM = 128
BATCH = 4096
CHUNK = 16384
NCHUNK = SIZE // CHUNK
NBUF = 7


def _body(ptr_ref, q_hbm, emb_hbm, out_hbm, optr_ref,
          bufs, emb_buf, sem_in, sem_out, sem_emb):
    raw_ptr = ptr_ref[0]
    ptr = jnp.clip(raw_ptr, 0, SIZE - BATCH)
    optr_ref[0] = (raw_ptr + BATCH) % SIZE

    k0 = ptr // CHUNK

    def in_copy(i):
        c = ((k0 + i) % NCHUNK) * CHUNK
        s = i % NBUF
        return pltpu.make_async_copy(
            q_hbm.at[pl.ds(c, CHUNK)], bufs.at[s], sem_in.at[s])

    def out_copy(i):
        c = ((k0 + i) % NCHUNK) * CHUNK
        s = i % NBUF
        return pltpu.make_async_copy(
            bufs.at[s], out_hbm.at[pl.ds(c, CHUNK)], sem_out.at[s])

    emb_in = pltpu.make_async_copy(emb_hbm, emb_buf, sem_emb)
    emb_out = pltpu.make_async_copy(
        emb_buf, out_hbm.at[pl.ds(ptr, BATCH)], sem_emb)

    emb_in.start()
    for s in range(NBUF):
        in_copy(s).start()

    out_waited = set()

    def ensure_out(j):
        if j not in out_waited:
            out_copy(j).wait()
            out_waited.add(j)

    for i in range(NCHUNK):
        in_copy(i).wait()
        out_copy(i).start()
        if i == 1:
            ensure_out(0)
            ensure_out(1)
            emb_in.wait()
            emb_out.start()
        nxt = i + NBUF
        if nxt < NCHUNK:
            ensure_out(nxt - NBUF)
            in_copy(nxt).start()
    for j in range(NCHUNK):
        ensure_out(j)
    emb_out.wait()


def kernel(embed_queue, queue_ptr, embeddings):
    new_queue, new_ptr = pl.pallas_call(
        _body,
        in_specs=[
            pl.BlockSpec(memory_space=pltpu.SMEM),
            pl.BlockSpec(memory_space=pl.ANY),
            pl.BlockSpec(memory_space=pl.ANY),
        ],
        out_specs=[
            pl.BlockSpec(memory_space=pl.ANY),
            pl.BlockSpec(memory_space=pltpu.SMEM),
        ],
        out_shape=[
            jax.ShapeDtypeStruct((SIZE, EMBED_DIM), jnp.float32),
            jax.ShapeDtypeStruct((1,), jnp.int32),
        ],
        scratch_shapes=[
            pltpu.VMEM((NBUF, CHUNK, EMBED_DIM), jnp.float32),
            pltpu.VMEM((BATCH, EMBED_DIM), jnp.float32),
            pltpu.SemaphoreType.DMA((NBUF,)),
            pltpu.SemaphoreType.DMA((NBUF,)),
            pltpu.SemaphoreType.DMA,
        ],
    )(queue_ptr, embed_queue, embeddings)
    return new_queue, new_ptr


# speedup vs baseline: 1.0194x; 1.0028x over previous
import jax
import jax.numpy as jnp
from jax.experimental import pallas as pl
from jax.experimental.pallas import tpu as pltpu

SIZE = 262144
EMBED_DIM = 128
BATCH = 4096
CHUNK = 16384
NCHUNK = SIZE // CHUNK
NBUF = 7


def _body(ptr_ref, q_hbm, emb_hbm, out_hbm, optr_ref,
          bufs, emb_buf, sem_in, sem_out, sem_emb):
    raw_ptr = ptr_ref[0]
    ptr = jnp.clip(raw_ptr, 0, SIZE - BATCH)
    optr_ref[0] = (raw_ptr + BATCH) % SIZE

    k0 = ptr // CHUNK

    def in_copy(i):
        c = ((k0 + i) % NCHUNK) * CHUNK
        s = i % NBUF
        return pltpu.make_async_copy(
            q_hbm.at[pl.ds(c, CHUNK)], bufs.at[pl.ds(s * CHUNK, CHUNK)],
            sem_in.at[s])

    def out_copy(i):
        c = ((k0 + i) % NCHUNK) * CHUNK
        s = i % NBUF
        return pltpu.make_async_copy(
            bufs.at[pl.ds(s * CHUNK, CHUNK)], out_hbm.at[pl.ds(c, CHUNK)],
            sem_out.at[s])

    emb_in = pltpu.make_async_copy(emb_hbm, emb_buf, sem_emb)
    off = ptr - k0 * CHUNK
    splice = pltpu.make_async_copy(
        emb_buf, bufs.at[pl.ds(off, BATCH)], sem_emb)

    emb_in.start()
    for s in range(NBUF):
        in_copy(s).start()

    in_waited = set()
    out_waited = set()

    def ensure_in(j):
        if j not in in_waited:
            in_copy(j).wait()
            in_waited.add(j)

    def ensure_out(j):
        if j not in out_waited:
            out_copy(j).wait()
            out_waited.add(j)

    ensure_in(0)
    ensure_in(1)
    emb_in.wait()
    splice.start()
    splice.wait()

    for i in range(NCHUNK):
        ensure_in(i)
        out_copy(i).start()
        nxt = i + NBUF
        if nxt < NCHUNK:
            ensure_out(nxt - NBUF)
            in_copy(nxt).start()
    for j in range(NCHUNK):
        ensure_out(j)


def kernel(embed_queue, queue_ptr, embeddings):
    new_queue, new_ptr = pl.pallas_call(
        _body,
        in_specs=[
            pl.BlockSpec(memory_space=pltpu.SMEM),
            pl.BlockSpec(memory_space=pl.ANY),
            pl.BlockSpec(memory_space=pl.ANY),
        ],
        out_specs=[
            pl.BlockSpec(memory_space=pl.ANY),
            pl.BlockSpec(memory_space=pltpu.SMEM),
        ],
        out_shape=[
            jax.ShapeDtypeStruct((SIZE, EMBED_DIM), jnp.float32),
            jax.ShapeDtypeStruct((1,), jnp.int32),
        ],
        scratch_shapes=[
            pltpu.VMEM((NBUF * CHUNK, EMBED_DIM), jnp.float32),
            pltpu.VMEM((BATCH, EMBED_DIM), jnp.float32),
            pltpu.SemaphoreType.DMA((NBUF,)),
            pltpu.SemaphoreType.DMA((NBUF,)),
            pltpu.SemaphoreType.DMA,
        ],
    )(queue_ptr, embed_queue, embeddings)
    return new_queue, new_ptr


# speedup vs baseline: 1.0262x; 1.0066x over previous
import jax
import jax.numpy as jnp
from jax.experimental import pallas as pl
from jax.experimental.pallas import tpu as pltpu

SIZE = 262144
EMBED_DIM = 128
BATCH = 4096
CHUNK = 16384
NCHUNK = SIZE // CHUNK
NBUF = 7


def _body(ptr_ref, q_hbm, emb_hbm, out_hbm, optr_ref,
          bufs, emb_buf, sem_in, sem_out, sem_emb):
    raw_ptr = ptr_ref[0]
    ptr = jnp.clip(raw_ptr, 0, SIZE - BATCH)
    optr_ref[0] = (raw_ptr + BATCH) % SIZE

    k0 = ptr // CHUNK

    ORDER = [2, 3, 0, 1] + list(range(4, NCHUNK))
    POS_OF_CHUNK0 = 2

    def in_copy(p):
        c = ((k0 + ORDER[p]) % NCHUNK) * CHUNK
        s = p % NBUF
        return pltpu.make_async_copy(
            q_hbm.at[pl.ds(c, CHUNK)], bufs.at[pl.ds(s * CHUNK, CHUNK)],
            sem_in.at[s])

    def out_copy(p):
        c = ((k0 + ORDER[p]) % NCHUNK) * CHUNK
        s = p % NBUF
        return pltpu.make_async_copy(
            bufs.at[pl.ds(s * CHUNK, CHUNK)], out_hbm.at[pl.ds(c, CHUNK)],
            sem_out.at[s])

    emb_in = pltpu.make_async_copy(emb_hbm, emb_buf, sem_emb)
    off = ptr - k0 * CHUNK
    splice = pltpu.make_async_copy(
        emb_buf, bufs.at[pl.ds(POS_OF_CHUNK0 * CHUNK + off, BATCH)], sem_emb)

    emb_in.start()
    for p in range(NBUF):
        in_copy(p).start()

    in_waited = set()
    out_waited = set()

    def ensure_in(p):
        if p not in in_waited:
            in_copy(p).wait()
            in_waited.add(p)

    def ensure_out(p):
        if p not in out_waited:
            out_copy(p).wait()
            out_waited.add(p)

    for p in range(NCHUNK):
        ensure_in(p)
        if p == POS_OF_CHUNK0:
            ensure_in(p + 1)
            emb_in.wait()
            splice.start()
            splice.wait()
        out_copy(p).start()
        nxt = p + NBUF
        if nxt < NCHUNK:
            ensure_out(nxt - NBUF)
            in_copy(nxt).start()
    for p in range(NCHUNK):
        ensure_out(p)


def kernel(embed_queue, queue_ptr, embeddings):
    new_queue, new_ptr = pl.pallas_call(
        _body,
        in_specs=[
            pl.BlockSpec(memory_space=pltpu.SMEM),
            pl.BlockSpec(memory_space=pl.ANY),
            pl.BlockSpec(memory_space=pl.ANY),
        ],
        out_specs=[
            pl.BlockSpec(memory_space=pl.ANY),
            pl.BlockSpec(memory_space=pltpu.SMEM),
        ],
        out_shape=[
            jax.ShapeDtypeStruct((SIZE, EMBED_DIM), jnp.float32),
            jax.ShapeDtypeStruct((1,), jnp.int32),
        ],
        scratch_shapes=[
            pltpu.VMEM((NBUF * CHUNK, EMBED_DIM), jnp.float32),
            pltpu.VMEM((BATCH, EMBED_DIM), jnp.float32),
            pltpu.SemaphoreType.DMA((NBUF,)),
            pltpu.SemaphoreType.DMA((NBUF,)),
            pltpu.SemaphoreType.DMA,
        ],
    )(queue_ptr, embed_queue, embeddings)
    return new_queue, new_ptr
